# Initial kernel scaffold; baseline (speedup 1.0000x reference)
import jax
import jax.numpy as jnp
from jax.experimental import pallas as pl


def kernel(x, cji, valence_mask, cutoff_w, rb, shb, idx_i, idx_j, tri_idx_k, edge_idx_kj, edge_idx_ji, W1, b1, W2, W3, W4, b4, W5, b5, W6, b6, W7):
    raise NotImplementedError("write your pallas kernel here")


# speedup vs baseline: 1.1856x; 1.1856x over previous
import functools

import jax
import jax.numpy as jnp
from jax.experimental import pallas as pl
from jax.experimental.pallas import tpu as pltpu

HID = 128
CO = 32
CV = 64
ORB = 8


def _silu(v):
    return v * jax.nn.sigmoid(v)


def _inv_norm(ssq):
    return jnp.where(ssq < 1e-24, 1e12, jax.lax.rsqrt(jnp.maximum(ssq, 1e-30)))


def _ka_body(x_ref, w1t_ref, b1_ref, xh_ref, sig_ref):
    h = jnp.dot(x_ref[...], w1t_ref[...], preferred_element_type=jnp.float32)
    h = h + b1_ref[...]
    xh_ref[...] = h[:, :CV]
    sig_ref[...] = jax.nn.sigmoid(h[:, CV:])


def _run_ka(x, W1, b1):
    n = x.shape[0]
    blk = 2000
    grid = n // blk
    return pl.pallas_call(
        _ka_body,
        grid=(grid,),
        in_specs=[
            pl.BlockSpec((blk, HID), lambda i: (i, 0)),
            pl.BlockSpec((HID, 2 * CV), lambda i: (0, 0)),
            pl.BlockSpec((1, 2 * CV), lambda i: (0, 0)),
        ],
        out_specs=[
            pl.BlockSpec((blk, CV), lambda i: (i, 0)),
            pl.BlockSpec((blk, CV), lambda i: (i, 0)),
        ],
        out_shape=[
            jax.ShapeDtypeStruct((n, CV), jnp.float32),
            jax.ShapeDtypeStruct((n, CV), jnp.float32),
        ],
        compiler_params=pltpu.CompilerParams(
            dimension_semantics=("parallel",)),
    )(x, W1.T, b1.reshape(1, 2 * CV))


def _kb_body(cji_ref, rbw_ref, w2t_ref, w3t_ref, cj_ref, p_ref):
    eb = cji_ref.shape[0]
    cf = cji_ref[...].reshape(eb * ORB, CO)
    c1 = jnp.dot(_silu(cf), w2t_ref[...], preferred_element_type=jnp.float32)
    c = jnp.dot(_silu(c1), w3t_ref[...], preferred_element_type=jnp.float32)
    c_ji = c[:, :CV]
    ckj = c[:, CV:]
    ssq = jnp.sum(ckj * ckj, axis=-1, keepdims=True)
    ckjn = ckj * _inv_norm(ssq)
    cj_ref[...] = c_ji.reshape(eb, ORB, CV)
    p_ref[...] = ckjn.reshape(eb, ORB, CV) * rbw_ref[...][:, :, None]


def _run_kb(cji, rb_w, W2, W3):
    e = cji.shape[0]
    blk = 1000
    grid = e // blk
    return pl.pallas_call(
        _kb_body,
        grid=(grid,),
        in_specs=[
            pl.BlockSpec((blk, ORB, CO), lambda i: (i, 0, 0)),
            pl.BlockSpec((blk, ORB), lambda i: (i, 0)),
            pl.BlockSpec((CO, CV), lambda i: (0, 0)),
            pl.BlockSpec((CV, 2 * CV), lambda i: (0, 0)),
        ],
        out_specs=[
            pl.BlockSpec((blk, ORB, CV), lambda i: (i, 0, 0)),
            pl.BlockSpec((blk, ORB, CV), lambda i: (i, 0, 0)),
        ],
        out_shape=[
            jax.ShapeDtypeStruct((e, ORB, CV), jnp.float32),
            jax.ShapeDtypeStruct((e, ORB, CV), jnp.float32),
        ],
        compiler_params=pltpu.CompilerParams(
            dimension_semantics=("parallel",)),
    )(cji, rb_w, W2.T, W3.T)


def _kc_body(tbwr_ref, cj_ref, rbw_ref, xhi_ref, xhj_ref,
             w4t_ref, b4_ref, w5at_ref, w5bt_ref, b5_ref, w6t_ref, b6_ref,
             msg_ref):
    tbw = jnp.dot(_silu(tbwr_ref[...]), w4t_ref[...],
                  preferred_element_type=jnp.float32) + b4_ref[...]
    c2 = cj_ref[...] * (1.0 + tbw[:, None, :])
    ssq2 = jnp.sum(c2 * c2, axis=-1, keepdims=True)
    c2 = c2 * _inv_norm(ssq2)
    lcao = jnp.sum(c2 * rbw_ref[...][:, :, None], axis=1)
    ssq3 = jnp.sum(lcao * lcao, axis=-1, keepdims=True)
    lcao = lcao * _inv_norm(ssq3)
    nf = (jnp.dot(_silu(xhi_ref[...]), w5at_ref[...],
                  preferred_element_type=jnp.float32)
          + jnp.dot(_silu(xhj_ref[...]), w5bt_ref[...],
                    preferred_element_type=jnp.float32) + b5_ref[...])
    nf = jnp.dot(_silu(nf), w6t_ref[...],
                 preferred_element_type=jnp.float32) + b6_ref[...]
    msg_ref[...] = lcao * nf


def _run_kc(tbw_raw, c_ji, rb_w, xhi, xhj, W4, b4, W5, b5, W6, b6):
    e = tbw_raw.shape[0]
    blk = 1000
    grid = e // blk
    return pl.pallas_call(
        _kc_body,
        grid=(grid,),
        in_specs=[
            pl.BlockSpec((blk, CV), lambda i: (i, 0)),
            pl.BlockSpec((blk, ORB, CV), lambda i: (i, 0, 0)),
            pl.BlockSpec((blk, ORB), lambda i: (i, 0)),
            pl.BlockSpec((blk, CV), lambda i: (i, 0)),
            pl.BlockSpec((blk, CV), lambda i: (i, 0)),
            pl.BlockSpec((CV, CV), lambda i: (0, 0)),
            pl.BlockSpec((1, CV), lambda i: (0, 0)),
            pl.BlockSpec((CV, CV), lambda i: (0, 0)),
            pl.BlockSpec((CV, CV), lambda i: (0, 0)),
            pl.BlockSpec((1, CV), lambda i: (0, 0)),
            pl.BlockSpec((CV, CV), lambda i: (0, 0)),
            pl.BlockSpec((1, CV), lambda i: (0, 0)),
        ],
        out_specs=pl.BlockSpec((blk, CV), lambda i: (i, 0)),
        out_shape=jax.ShapeDtypeStruct((e, CV), jnp.float32),
        compiler_params=pltpu.CompilerParams(
            dimension_semantics=("parallel",)),
    )(tbw_raw, c_ji, rb_w, xhi, xhj,
      W4.T, b4.reshape(1, CV), W5[:, :CV].T, W5[:, CV:].T, b5.reshape(1, CV),
      W6.T, b6.reshape(1, CV))


def _kd_body(x_ref, agg_ref, w7t_ref, out_ref):
    out_ref[...] = x_ref[...] + jnp.dot(
        agg_ref[...], w7t_ref[...], preferred_element_type=jnp.float32)


def _run_kd(x, agg, W7):
    n = x.shape[0]
    blk = 2000
    grid = n // blk
    return pl.pallas_call(
        _kd_body,
        grid=(grid,),
        in_specs=[
            pl.BlockSpec((blk, HID), lambda i: (i, 0)),
            pl.BlockSpec((blk, CV), lambda i: (i, 0)),
            pl.BlockSpec((CV, HID), lambda i: (0, 0)),
        ],
        out_specs=pl.BlockSpec((blk, HID), lambda i: (i, 0)),
        out_shape=jax.ShapeDtypeStruct((n, HID), jnp.float32),
        compiler_params=pltpu.CompilerParams(
            dimension_semantics=("parallel",)),
    )(x, agg, W7.T)


def kernel(x, cji, valence_mask, cutoff_w, rb, shb, idx_i, idx_j, tri_idx_k,
           edge_idx_kj, edge_idx_ji, W1, b1, W2, W3, W4, b4, W5, b5, W6, b6,
           W7):
    e = rb.shape[0]
    n = x.shape[0]
    rb_w = rb * cutoff_w[:, None]

    xh, sigxk = _run_ka(x, W1, b1)
    c_ji, P = _run_kb(cji, rb_w, W2, W3)

    tbo = jnp.einsum('td,tdh->th', shb, P[edge_idx_kj])
    ssq_t = jnp.sum(tbo * tbo, axis=-1, keepdims=True)
    contrib = tbo * _inv_norm(ssq_t) * sigxk[tri_idx_k]
    tbw_raw = jax.ops.segment_sum(contrib, edge_idx_ji, num_segments=e)

    xhi = xh[idx_i]
    xhj = xh[idx_j]
    msg = _run_kc(tbw_raw, c_ji, rb_w, xhi, xhj, W4, b4, W5, b5, W6, b6)
    agg = jax.ops.segment_sum(msg, idx_i, num_segments=n)
    return _run_kd(x, agg, W7)
